# baseline (speedup 1.0000x reference)
import math

import jax
import jax.numpy as jnp
from jax import lax
from jax.experimental import pallas as pl

_N_PER = 32
_E_PER = 64
_WL = 20
_NL = 3
_HEADS = 4
_G = 8
_NPG = _G * _N_PER
_EPG = _G * _E_PER
_RSBN = 1.0 / math.sqrt(1.0 + 1e-5)


def _fused_kernel(x_ref, ea_ref, src_ref, dst_ref, mask_ref, eye_ref, eyeb_ref,
                  Wi_ref, bi_ref, We_ref, be_ref,
                  Wg1_ref, bg1_ref, Wg2_ref, bg2_ref,
                  Wqkv_ref, bqkv_ref, Wo_ref, bo_ref,
                  Wm1_ref, bm1_ref, Wm2_ref, bm2_ref,
                  Wh1_ref, bh1_ref, Wh2t_ref, bh2_ref, out_ref):
    f32 = jnp.float32
    bf16 = jnp.bfloat16
    hid = Wi_ref.shape[1]
    dh = hid // _HEADS

    x = x_ref[...]
    ea = ea_ref[...]
    src = src_ref[...]
    dst = dst_ref[...]
    mask = mask_ref[...]
    eye = eye_ref[...]
    eye_b = eyeb_ref[...]

    n_iota = lax.broadcasted_iota(jnp.int32, (_G, _E_PER, _NPG), 2)
    soh_b = (src[:, :, None] == n_iota).astype(bf16).reshape(_EPG, _NPG)
    doh_b = (dst[:, :, None] == n_iota).astype(bf16).reshape(_EPG, _NPG)

    deg = jnp.sum(soh_b, axis=0, keepdims=True).astype(f32) + 1.0
    inv = 1.0 / deg
    m_raw = jnp.dot(doh_b.T, soh_b, preferred_element_type=f32)
    M = (m_raw + eye) * inv

    def diag_row(p_b):
        return jnp.sum(p_b * eye_b, axis=0, keepdims=True).astype(f32)

    Mb = M.astype(bf16)
    M2b = jnp.dot(Mb, Mb, preferred_element_type=f32).astype(bf16)
    M3b = jnp.dot(M2b, Mb, preferred_element_type=f32).astype(bf16)
    M4b = jnp.dot(M2b, M2b, preferred_element_type=f32).astype(bf16)
    cols = [None] * _WL
    cols[0], cols[1], cols[2], cols[3] = (
        diag_row(Mb), diag_row(M2b), diag_row(M3b), diag_row(M4b))
    seeds = [Mb, M2b, M3b, M4b]
    for r in range(4):
        q = seeds[r]
        for s in range(1, 5):
            t = r + 4 * s
            if t >= _WL:
                break
            q = jnp.dot(q, M4b, preferred_element_type=f32).astype(bf16)
            cols[t] = diag_row(q)
    pe_t = jnp.concatenate(cols + [jnp.zeros((12, _NPG), f32)], axis=0)
    pe = pe_t.T[:, :_WL]

    h = jnp.dot(jnp.concatenate([x, pe.astype(bf16)], axis=1), Wi_ref[...],
                preferred_element_type=f32) + bi_ref[...]
    ee = jnp.dot(ea, We_ref[...], preferred_element_type=f32) + be_ref[...]

    for l in range(_NL):
        hb = h.astype(bf16)
        msg = jnp.maximum(
            jnp.dot(soh_b, hb, preferred_element_type=f32) + ee, 0.0)
        agg = jnp.dot(doh_b.T, msg.astype(bf16), preferred_element_type=f32)
        hl = h + agg
        hl = jnp.dot(
            jnp.maximum(jnp.dot(hl.astype(bf16), Wg1_ref[l],
                                preferred_element_type=f32)
                        + bg1_ref[l][None, :], 0.0).astype(bf16),
            Wg2_ref[l], preferred_element_type=f32) + bg2_ref[l][None, :]
        hl = (hl + h) * _RSBN

        qkv = jnp.dot(hb, Wqkv_ref[l], preferred_element_type=f32) \
            + bqkv_ref[l][None, :]
        heads = []
        for t in range(_HEADS):
            q = qkv[:, t * dh:(t + 1) * dh].astype(bf16)
            k = qkv[:, hid + t * dh:hid + (t + 1) * dh].astype(bf16)
            v = qkv[:, 2 * hid + t * dh:2 * hid + (t + 1) * dh].astype(bf16)
            sc = jnp.dot(q, k.T, preferred_element_type=f32) + mask
            sc = sc - jnp.max(sc, axis=1, keepdims=True)
            e = jnp.exp(sc)
            rinv = 1.0 / jnp.sum(e, axis=1, keepdims=True)
            heads.append(
                jnp.dot(e.astype(bf16), v, preferred_element_type=f32) * rinv)
        att = jnp.concatenate(heads, axis=1).astype(bf16)
        ha = jnp.dot(att, Wo_ref[l], preferred_element_type=f32) \
            + bo_ref[l][None, :]
        ha = (ha + h) * _RSBN

        out = hl + ha
        mlp = jnp.dot(
            jnp.maximum(jnp.dot(out.astype(bf16), Wm1_ref[l],
                                preferred_element_type=f32)
                        + bm1_ref[l][None, :], 0.0).astype(bf16),
            Wm2_ref[l], preferred_element_type=f32) + bm2_ref[l][None, :]
        h = (out + mlp) * _RSBN

    pooled = jnp.mean(h.reshape(_G, _N_PER, hid), axis=1)
    r = jnp.maximum(jnp.dot(pooled, Wh1_ref[...], preferred_element_type=f32)
                    + bh1_ref[...], 0.0)
    y = jnp.sum(r * Wh2t_ref[...], axis=1, keepdims=True)
    out_ref[...] = jnp.broadcast_to(y, out_ref.shape) + bh2_ref[...]


def kernel(x, edge_attr, Wi, bi, We, be, Wg1, bg1, Wg2, bg2, Wqkv, bqkv,
           Wo, bo, Wm1, bm1, Wm2, bm2, Wh1, bh1, Wh2, bh2, edge_index, batch):
    n, d_in = x.shape
    hid = Wi.shape[1]
    n_mol = n // _N_PER
    grid = n_mol // _G

    src2 = (edge_index[0] % _NPG).astype(jnp.int32).reshape(n_mol, _E_PER)
    dst2 = (edge_index[1] % _NPG).astype(jnp.int32).reshape(n_mol, _E_PER)
    scale = 1.0 / math.sqrt(hid // _HEADS)
    Wqkv = jnp.concatenate([Wqkv[:, :, :hid] * scale, Wqkv[:, :, hid:]], axis=2)
    bqkv = jnp.concatenate([bqkv[:, :hid] * scale, bqkv[:, hid:]], axis=1)

    mol_id = jnp.arange(_NPG, dtype=jnp.int32) // _N_PER
    mask = jnp.where(mol_id[:, None] == mol_id[None, :], 0.0, -1e30)
    mask = mask.astype(jnp.float32)
    eye = jnp.eye(_NPG, dtype=jnp.float32)

    const2 = lambda s: pl.BlockSpec(s, lambda i: (0, 0))
    const3 = lambda s: pl.BlockSpec(s, lambda i: (0, 0, 0))

    out = pl.pallas_call(
        _fused_kernel,
        grid=(grid,),
        in_specs=[
            pl.BlockSpec((_NPG, d_in), lambda i: (i, 0)),
            pl.BlockSpec((_EPG, edge_attr.shape[1]), lambda i: (i, 0)),
            pl.BlockSpec((_G, _E_PER), lambda i: (i, 0)),
            pl.BlockSpec((_G, _E_PER), lambda i: (i, 0)),
            const2(mask.shape), const2(eye.shape), const2(eye.shape),
            const2(Wi.shape), const2((1, hid)),
            const2(We.shape), const2((1, hid)),
            const3(Wg1.shape), const2(bg1.shape),
            const3(Wg2.shape), const2(bg2.shape),
            const3(Wqkv.shape), const2(bqkv.shape),
            const3(Wo.shape), const2(bo.shape),
            const3(Wm1.shape), const2(bm1.shape),
            const3(Wm2.shape), const2(bm2.shape),
            const2(Wh1.shape), const2((1, Wh1.shape[1])),
            const2((1, Wh2.shape[0])),
            const2((1, hid)),
        ],
        out_specs=pl.BlockSpec((_G, hid), lambda i: (i, 0)),
        out_shape=jax.ShapeDtypeStruct((n_mol, hid), jnp.float32),
    )(
        x.astype(jnp.bfloat16), edge_attr.astype(jnp.bfloat16),
        src2, dst2, mask, eye, eye.astype(jnp.bfloat16),
        Wi.astype(jnp.bfloat16), bi.reshape(1, -1),
        We.astype(jnp.bfloat16), be.reshape(1, -1),
        Wg1.astype(jnp.bfloat16), bg1, Wg2.astype(jnp.bfloat16), bg2,
        Wqkv.astype(jnp.bfloat16), bqkv, Wo.astype(jnp.bfloat16), bo,
        Wm1.astype(jnp.bfloat16), bm1, Wm2.astype(jnp.bfloat16), bm2,
        Wh1, bh1.reshape(1, -1), Wh2.reshape(1, -1),
        jnp.broadcast_to(bh2.reshape(1, 1), (1, hid)),
    )
    return out[:, 0]


# speedup vs baseline: 1.0299x; 1.0299x over previous
import math

import jax
import jax.numpy as jnp
from jax import lax
from jax.experimental import pallas as pl

_N_PER = 32
_E_PER = 64
_WL = 20
_NL = 3
_HEADS = 4
_G = 8
_NS = 2
_GS = _G * _NS
_NPG = _G * _N_PER
_EPG = _G * _E_PER
_RSBN = 1.0 / math.sqrt(1.0 + 1e-5)


def _fused_kernel(x_ref, ea_ref, src_ref, dst_ref, mask_ref, eye_ref, eyeb_ref,
                  Wi_ref, bi_ref, We_ref, be_ref,
                  Wg1_ref, bg1_ref, Wg2_ref, bg2_ref,
                  Wqkv_ref, bqkv_ref, Wo_ref, bo_ref,
                  Wm1_ref, bm1_ref, Wm2_ref, bm2_ref,
                  Wh1_ref, bh1_ref, Wh2t_ref, bh2_ref, out_ref):
    f32 = jnp.float32
    bf16 = jnp.bfloat16
    hid = Wi_ref.shape[1]
    dh = hid // _HEADS

    xs = x_ref[...]
    eas = ea_ref[...]
    srcs = src_ref[...]
    dsts = dst_ref[...]
    mask = mask_ref[...]
    eye = eye_ref[...]
    eye_b = eyeb_ref[...]

    def _stream(x, ea, src, dst):
        n_iota = lax.broadcasted_iota(jnp.int32, (_G, _E_PER, _NPG), 2)
        soh_b = (src[:, :, None] == n_iota).astype(bf16).reshape(_EPG, _NPG)
        doh_b = (dst[:, :, None] == n_iota).astype(bf16).reshape(_EPG, _NPG)

        deg = jnp.sum(soh_b, axis=0, keepdims=True).astype(f32) + 1.0
        inv = 1.0 / deg
        m_raw = jnp.dot(doh_b.T, soh_b, preferred_element_type=f32)
        M = (m_raw + eye) * inv

        def diag_row(p_b):
            return jnp.sum(p_b * eye_b, axis=0, keepdims=True).astype(f32)

        Mb = M.astype(bf16)
        M2b = jnp.dot(Mb, Mb, preferred_element_type=f32).astype(bf16)
        M3b = jnp.dot(M2b, Mb, preferred_element_type=f32).astype(bf16)
        M4b = jnp.dot(M2b, M2b, preferred_element_type=f32).astype(bf16)
        cols = [None] * _WL
        cols[0], cols[1], cols[2], cols[3] = (
            diag_row(Mb), diag_row(M2b), diag_row(M3b), diag_row(M4b))
        seeds = [Mb, M2b, M3b, M4b]
        for r in range(4):
            q = seeds[r]
            for s in range(1, 5):
                t = r + 4 * s
                if t >= _WL:
                    break
                q = jnp.dot(q, M4b, preferred_element_type=f32).astype(bf16)
                cols[t] = diag_row(q)
        pe_t = jnp.concatenate(cols + [jnp.zeros((12, _NPG), f32)], axis=0)
        pe = pe_t.T[:, :_WL]

        h = jnp.dot(jnp.concatenate([x, pe.astype(bf16)], axis=1), Wi_ref[...],
                    preferred_element_type=f32) + bi_ref[...]
        ee = jnp.dot(ea, We_ref[...], preferred_element_type=f32) + be_ref[...]

        for l in range(_NL):
            hb = h.astype(bf16)
            msg = jnp.maximum(
                jnp.dot(soh_b, hb, preferred_element_type=f32) + ee, 0.0)
            agg = jnp.dot(doh_b.T, msg.astype(bf16), preferred_element_type=f32)
            hl = h + agg
            hl = jnp.dot(
                jnp.maximum(jnp.dot(hl.astype(bf16), Wg1_ref[l],
                                    preferred_element_type=f32)
                            + bg1_ref[l][None, :], 0.0).astype(bf16),
                Wg2_ref[l], preferred_element_type=f32) + bg2_ref[l][None, :]
            hl = (hl + h) * _RSBN

            qkv = jnp.dot(hb, Wqkv_ref[l], preferred_element_type=f32) \
                + bqkv_ref[l][None, :]
            heads = []
            for t in range(_HEADS):
                q = qkv[:, t * dh:(t + 1) * dh].astype(bf16)
                k = qkv[:, hid + t * dh:hid + (t + 1) * dh].astype(bf16)
                v = qkv[:, 2 * hid + t * dh:2 * hid + (t + 1) * dh].astype(bf16)
                sc = jnp.dot(q, k.T, preferred_element_type=f32) + mask
                sc = sc - jnp.max(sc, axis=1, keepdims=True)
                e = jnp.exp(sc)
                rinv = 1.0 / jnp.sum(e, axis=1, keepdims=True)
                heads.append(
                    jnp.dot(e.astype(bf16), v, preferred_element_type=f32) * rinv)
            att = jnp.concatenate(heads, axis=1).astype(bf16)
            ha = jnp.dot(att, Wo_ref[l], preferred_element_type=f32) \
                + bo_ref[l][None, :]
            ha = (ha + h) * _RSBN

            out = hl + ha
            mlp = jnp.dot(
                jnp.maximum(jnp.dot(out.astype(bf16), Wm1_ref[l],
                                    preferred_element_type=f32)
                            + bm1_ref[l][None, :], 0.0).astype(bf16),
                Wm2_ref[l], preferred_element_type=f32) + bm2_ref[l][None, :]
            h = (out + mlp) * _RSBN
        return h

    hs = jnp.concatenate(
        [_stream(xs[i * _NPG:(i + 1) * _NPG],
                 eas[i * _EPG:(i + 1) * _EPG],
                 srcs[i * _G:(i + 1) * _G],
                 dsts[i * _G:(i + 1) * _G]) for i in range(_NS)], axis=0)

    pooled = jnp.mean(hs.reshape(_GS, _N_PER, hid), axis=1)
    r = jnp.maximum(jnp.dot(pooled, Wh1_ref[...], preferred_element_type=f32)
                    + bh1_ref[...], 0.0)
    y = jnp.sum(r * Wh2t_ref[...], axis=1, keepdims=True)
    out_ref[...] = jnp.broadcast_to(y, out_ref.shape) + bh2_ref[...]


def kernel(x, edge_attr, Wi, bi, We, be, Wg1, bg1, Wg2, bg2, Wqkv, bqkv,
           Wo, bo, Wm1, bm1, Wm2, bm2, Wh1, bh1, Wh2, bh2, edge_index, batch):
    n, d_in = x.shape
    hid = Wi.shape[1]
    n_mol = n // _N_PER
    grid = n_mol // _GS

    src2 = (edge_index[0] % _NPG).astype(jnp.int32).reshape(n_mol, _E_PER)
    dst2 = (edge_index[1] % _NPG).astype(jnp.int32).reshape(n_mol, _E_PER)
    scale = 1.0 / math.sqrt(hid // _HEADS)
    Wqkv = jnp.concatenate([Wqkv[:, :, :hid] * scale, Wqkv[:, :, hid:]], axis=2)
    bqkv = jnp.concatenate([bqkv[:, :hid] * scale, bqkv[:, hid:]], axis=1)

    mol_id = jnp.arange(_NPG, dtype=jnp.int32) // _N_PER
    mask = jnp.where(mol_id[:, None] == mol_id[None, :], 0.0, -1e30)
    mask = mask.astype(jnp.float32)
    eye = jnp.eye(_NPG, dtype=jnp.float32)

    const2 = lambda s: pl.BlockSpec(s, lambda i: (0, 0))
    const3 = lambda s: pl.BlockSpec(s, lambda i: (0, 0, 0))

    out = pl.pallas_call(
        _fused_kernel,
        grid=(grid,),
        in_specs=[
            pl.BlockSpec((_NS * _NPG, d_in), lambda i: (i, 0)),
            pl.BlockSpec((_NS * _EPG, edge_attr.shape[1]), lambda i: (i, 0)),
            pl.BlockSpec((_GS, _E_PER), lambda i: (i, 0)),
            pl.BlockSpec((_GS, _E_PER), lambda i: (i, 0)),
            const2(mask.shape), const2(eye.shape), const2(eye.shape),
            const2(Wi.shape), const2((1, hid)),
            const2(We.shape), const2((1, hid)),
            const3(Wg1.shape), const2(bg1.shape),
            const3(Wg2.shape), const2(bg2.shape),
            const3(Wqkv.shape), const2(bqkv.shape),
            const3(Wo.shape), const2(bo.shape),
            const3(Wm1.shape), const2(bm1.shape),
            const3(Wm2.shape), const2(bm2.shape),
            const2(Wh1.shape), const2((1, Wh1.shape[1])),
            const2((1, Wh2.shape[0])),
            const2((1, hid)),
        ],
        out_specs=pl.BlockSpec((_GS, hid), lambda i: (i, 0)),
        out_shape=jax.ShapeDtypeStruct((n_mol, hid), jnp.float32),
    )(
        x.astype(jnp.bfloat16), edge_attr.astype(jnp.bfloat16),
        src2, dst2, mask, eye, eye.astype(jnp.bfloat16),
        Wi.astype(jnp.bfloat16), bi.reshape(1, -1),
        We.astype(jnp.bfloat16), be.reshape(1, -1),
        Wg1.astype(jnp.bfloat16), bg1, Wg2.astype(jnp.bfloat16), bg2,
        Wqkv.astype(jnp.bfloat16), bqkv, Wo.astype(jnp.bfloat16), bo,
        Wm1.astype(jnp.bfloat16), bm1, Wm2.astype(jnp.bfloat16), bm2,
        Wh1, bh1.reshape(1, -1), Wh2.reshape(1, -1),
        jnp.broadcast_to(bh2.reshape(1, 1), (1, hid)),
    )
    return out[:, 0]


# speedup vs baseline: 1.2269x; 1.1913x over previous
import math

import jax
import jax.numpy as jnp
from jax import lax
from jax.experimental import pallas as pl

_N_PER = 32
_E_PER = 64
_WL = 20
_NL = 3
_HEADS = 4
_G = 8
_NS = 2
_GS = _G * _NS
_NPG = _G * _N_PER
_EPG = _G * _E_PER
_RSBN = 1.0 / math.sqrt(1.0 + 1e-5)


def _fused_kernel(x_ref, ea_ref, src_ref, dst_ref, mask_ref, eye_ref,
                  eyeb_ref,
                  Wi_ref, bi_ref, We_ref, be_ref,
                  Wg1_ref, bg1_ref, Wg2_ref, bg2_ref,
                  Wqkv_ref, bqkv_ref, Wo_ref, bo_ref,
                  Wm1_ref, bm1_ref, Wm2_ref, bm2_ref,
                  Wh1_ref, bh1_ref, Wh2t_ref, bh2_ref, out_ref):
    f32 = jnp.float32
    bf16 = jnp.bfloat16
    hid = Wi_ref.shape[1]
    dh = hid // _HEADS
    R = range(_NS)

    xs = x_ref[...]
    eas = ea_ref[...]
    srcs = src_ref[...]
    dsts = dst_ref[...]
    mask = mask_ref[...]
    eye = eye_ref[...]
    eye_b = eyeb_ref[...]

    def nrows(a, i):
        return a[i * _NPG:(i + 1) * _NPG]

    def erows(a, i):
        return a[i * _EPG:(i + 1) * _EPG]

    n_iota = lax.broadcasted_iota(jnp.int32, (_G, _E_PER, _NPG), 2)

    def onehot(idx):
        oh = (idx[:, :, None] == n_iota).astype(bf16)
        return oh.reshape(_EPG, _NPG)

    soh = [onehot(srcs[i * _G:(i + 1) * _G]) for i in R]
    doh = [onehot(dsts[i * _G:(i + 1) * _G]) for i in R]

    deg = [jnp.sum(soh[i], axis=0, keepdims=True).astype(f32) + 1.0 for i in R]
    inv = [1.0 / deg[i] for i in R]
    m_raw = [jnp.dot(doh[i].T, soh[i], preferred_element_type=f32) for i in R]
    Mb = [((m_raw[i] + eye) * inv[i]).astype(bf16) for i in R]

    def diag_row(p_b):
        return jnp.sum(p_b * eye_b, axis=0, keepdims=True).astype(f32)

    M2b = [jnp.dot(Mb[i], Mb[i], preferred_element_type=f32).astype(bf16)
           for i in R]
    M3b = [jnp.dot(M2b[i], Mb[i], preferred_element_type=f32).astype(bf16)
           for i in R]
    M4b = [jnp.dot(M2b[i], M2b[i], preferred_element_type=f32).astype(bf16)
           for i in R]
    cols = [[None] * _WL for _ in R]
    chain = [[Mb[i], M2b[i], M3b[i], M4b[i]] for i in R]
    for i in R:
        for r in range(4):
            cols[i][r] = diag_row(chain[i][r])
    for s in range(1, 5):
        for r in range(4):
            t = r + 4 * s
            if t >= _WL:
                continue
            for i in R:
                chain[i][r] = jnp.dot(chain[i][r], M4b[i],
                                      preferred_element_type=f32).astype(bf16)
                cols[i][t] = diag_row(chain[i][r])
    pad = jnp.zeros((12, _NPG), f32)
    pe = [jnp.concatenate(cols[i] + [pad], axis=0).T[:, :_WL] for i in R]

    pe_all = jnp.concatenate(pe, axis=0).astype(bf16)
    h = jnp.dot(jnp.concatenate([xs, pe_all], axis=1), Wi_ref[...],
                preferred_element_type=f32) + bi_ref[...]
    ee = jnp.dot(eas, We_ref[...], preferred_element_type=f32) + be_ref[...]

    for l in range(_NL):
        hb = h.astype(bf16)
        msg = [jnp.maximum(
            jnp.dot(soh[i], nrows(hb, i), preferred_element_type=f32)
            + erows(ee, i), 0.0).astype(bf16) for i in R]
        agg = [jnp.dot(doh[i].T, msg[i], preferred_element_type=f32)
               for i in R]
        hl = h + jnp.concatenate(agg, axis=0)
        hl = jnp.dot(
            jnp.maximum(jnp.dot(hl.astype(bf16), Wg1_ref[l],
                                preferred_element_type=f32)
                        + bg1_ref[l][None, :], 0.0).astype(bf16),
            Wg2_ref[l], preferred_element_type=f32) + bg2_ref[l][None, :]
        hl = (hl + h) * _RSBN

        qkv = jnp.dot(hb, Wqkv_ref[l], preferred_element_type=f32) \
            + bqkv_ref[l][None, :]
        heads = [[None] * _HEADS for _ in R]
        for t in range(_HEADS):
            for i in R:
                qi = nrows(qkv, i)
                q = qi[:, t * dh:(t + 1) * dh].astype(bf16)
                k = qi[:, hid + t * dh:hid + (t + 1) * dh].astype(bf16)
                v = qi[:, 2 * hid + t * dh:2 * hid + (t + 1) * dh].astype(bf16)
                sc = jnp.dot(q, k.T, preferred_element_type=f32) + mask
                sc = sc - jnp.max(sc, axis=1, keepdims=True)
                e = jnp.exp(sc)
                rinv = 1.0 / jnp.sum(e, axis=1, keepdims=True)
                heads[i][t] = jnp.dot(e.astype(bf16), v,
                                      preferred_element_type=f32) * rinv
        att = jnp.concatenate(
            [jnp.concatenate(heads[i], axis=1) for i in R], axis=0)
        ha = jnp.dot(att.astype(bf16), Wo_ref[l],
                     preferred_element_type=f32) + bo_ref[l][None, :]
        ha = (ha + h) * _RSBN

        out = hl + ha
        mlp = jnp.dot(
            jnp.maximum(jnp.dot(out.astype(bf16), Wm1_ref[l],
                                preferred_element_type=f32)
                        + bm1_ref[l][None, :], 0.0).astype(bf16),
            Wm2_ref[l], preferred_element_type=f32) + bm2_ref[l][None, :]
        h = (out + mlp) * _RSBN

    pooled = jnp.mean(h.reshape(_GS, _N_PER, hid), axis=1)
    r = jnp.maximum(jnp.dot(pooled, Wh1_ref[...], preferred_element_type=f32)
                    + bh1_ref[...], 0.0)
    y = jnp.sum(r * Wh2t_ref[...], axis=1, keepdims=True)
    out_ref[...] = jnp.broadcast_to(y, out_ref.shape) + bh2_ref[...]


def kernel(x, edge_attr, Wi, bi, We, be, Wg1, bg1, Wg2, bg2, Wqkv, bqkv,
           Wo, bo, Wm1, bm1, Wm2, bm2, Wh1, bh1, Wh2, bh2, edge_index, batch):
    n, d_in = x.shape
    hid = Wi.shape[1]
    n_mol = n // _N_PER
    grid = n_mol // _GS

    src2 = (edge_index[0] % _NPG).astype(jnp.int32).reshape(n_mol, _E_PER)
    dst2 = (edge_index[1] % _NPG).astype(jnp.int32).reshape(n_mol, _E_PER)
    scale = 1.0 / math.sqrt(hid // _HEADS)
    Wqkv = jnp.concatenate([Wqkv[:, :, :hid] * scale, Wqkv[:, :, hid:]],
                           axis=2)
    bqkv = jnp.concatenate([bqkv[:, :hid] * scale, bqkv[:, hid:]], axis=1)

    mol_id = jnp.arange(_NPG, dtype=jnp.int32) // _N_PER
    mask = jnp.where(mol_id[:, None] == mol_id[None, :], 0.0, -1e30)
    mask = mask.astype(jnp.float32)
    eye = jnp.eye(_NPG, dtype=jnp.float32)

    const2 = lambda s: pl.BlockSpec(s, lambda i: (0, 0))
    const3 = lambda s: pl.BlockSpec(s, lambda i: (0, 0, 0))

    out = pl.pallas_call(
        _fused_kernel,
        grid=(grid,),
        in_specs=[
            pl.BlockSpec((_NS * _NPG, d_in), lambda i: (i, 0)),
            pl.BlockSpec((_NS * _EPG, edge_attr.shape[1]), lambda i: (i, 0)),
            pl.BlockSpec((_GS, _E_PER), lambda i: (i, 0)),
            pl.BlockSpec((_GS, _E_PER), lambda i: (i, 0)),
            const2(mask.shape), const2(eye.shape), const2(eye.shape),
            const2(Wi.shape), const2((1, hid)),
            const2(We.shape), const2((1, hid)),
            const3(Wg1.shape), const2(bg1.shape),
            const3(Wg2.shape), const2(bg2.shape),
            const3(Wqkv.shape), const2(bqkv.shape),
            const3(Wo.shape), const2(bo.shape),
            const3(Wm1.shape), const2(bm1.shape),
            const3(Wm2.shape), const2(bm2.shape),
            const2(Wh1.shape), const2((1, Wh1.shape[1])),
            const2((1, Wh2.shape[0])),
            const2((1, hid)),
        ],
        out_specs=pl.BlockSpec((_GS, hid), lambda i: (i, 0)),
        out_shape=jax.ShapeDtypeStruct((n_mol, hid), jnp.float32),
    )(
        x.astype(jnp.bfloat16), edge_attr.astype(jnp.bfloat16),
        src2, dst2, mask, eye, eye.astype(jnp.bfloat16),
        Wi.astype(jnp.bfloat16), bi.reshape(1, -1),
        We.astype(jnp.bfloat16), be.reshape(1, -1),
        Wg1.astype(jnp.bfloat16), bg1, Wg2.astype(jnp.bfloat16), bg2,
        Wqkv.astype(jnp.bfloat16), bqkv, Wo.astype(jnp.bfloat16), bo,
        Wm1.astype(jnp.bfloat16), bm1, Wm2.astype(jnp.bfloat16), bm2,
        Wh1, bh1.reshape(1, -1), Wh2.reshape(1, -1),
        jnp.broadcast_to(bh2.reshape(1, 1), (1, hid)),
    )
    return out[:, 0]


# speedup vs baseline: 1.3225x; 1.0779x over previous
import math

import jax
import jax.numpy as jnp
from jax import lax
from jax.experimental import pallas as pl

_N_PER = 32
_E_PER = 64
_WL = 20
_NL = 3
_HEADS = 4
_G = 8
_NS = 4
_GS = _G * _NS
_NPG = _G * _N_PER
_EPG = _G * _E_PER
_RSBN = 1.0 / math.sqrt(1.0 + 1e-5)


def _fused_kernel(x_ref, ea_ref, src_ref, dst_ref, mask_ref, eye_ref,
                  eyeb_ref,
                  Wi_ref, bi_ref, We_ref, be_ref,
                  Wg1_ref, bg1_ref, Wg2_ref, bg2_ref,
                  Wqkv_ref, bqkv_ref, Wo_ref, bo_ref,
                  Wm1_ref, bm1_ref, Wm2_ref, bm2_ref,
                  Wh1_ref, bh1_ref, Wh2t_ref, bh2_ref, out_ref):
    f32 = jnp.float32
    bf16 = jnp.bfloat16
    hid = Wi_ref.shape[1]
    dh = hid // _HEADS
    R = range(_NS)

    xs = x_ref[...]
    eas = ea_ref[...]
    srcs = src_ref[...]
    dsts = dst_ref[...]
    mask = mask_ref[...]
    eye = eye_ref[...]
    eye_b = eyeb_ref[...]

    def nrows(a, i):
        return a[i * _NPG:(i + 1) * _NPG]

    def erows(a, i):
        return a[i * _EPG:(i + 1) * _EPG]

    n_iota = lax.broadcasted_iota(jnp.int32, (_G, _E_PER, _NPG), 2)

    def onehot(idx):
        oh = (idx[:, :, None] == n_iota).astype(bf16)
        return oh.reshape(_EPG, _NPG)

    soh = [onehot(srcs[i * _G:(i + 1) * _G]) for i in R]
    doh = [onehot(dsts[i * _G:(i + 1) * _G]) for i in R]

    deg = [jnp.sum(soh[i], axis=0, keepdims=True).astype(f32) + 1.0 for i in R]
    inv = [1.0 / deg[i] for i in R]
    m_raw = [jnp.dot(doh[i].T, soh[i], preferred_element_type=f32) for i in R]
    Mb = [((m_raw[i] + eye) * inv[i]).astype(bf16) for i in R]

    def diag_row(p_b):
        return jnp.sum(p_b * eye_b, axis=0, keepdims=True).astype(f32)

    M2b = [jnp.dot(Mb[i], Mb[i], preferred_element_type=f32).astype(bf16)
           for i in R]
    M3b = [jnp.dot(M2b[i], Mb[i], preferred_element_type=f32).astype(bf16)
           for i in R]
    M4b = [jnp.dot(M2b[i], M2b[i], preferred_element_type=f32).astype(bf16)
           for i in R]
    cols = [[None] * _WL for _ in R]
    chain = [[Mb[i], M2b[i], M3b[i], M4b[i]] for i in R]
    for i in R:
        for r in range(4):
            cols[i][r] = diag_row(chain[i][r])
    for s in range(1, 5):
        for r in range(4):
            t = r + 4 * s
            if t >= _WL:
                continue
            for i in R:
                chain[i][r] = jnp.dot(chain[i][r], M4b[i],
                                      preferred_element_type=f32).astype(bf16)
                cols[i][t] = diag_row(chain[i][r])
    pad = jnp.zeros((12, _NPG), f32)
    pe = [jnp.concatenate(cols[i] + [pad], axis=0).T[:, :_WL] for i in R]

    pe_all = jnp.concatenate(pe, axis=0).astype(bf16)
    h = jnp.dot(jnp.concatenate([xs, pe_all], axis=1), Wi_ref[...],
                preferred_element_type=f32) + bi_ref[...]
    ee = jnp.dot(eas, We_ref[...], preferred_element_type=f32) + be_ref[...]

    for l in range(_NL):
        hb = h.astype(bf16)
        msg = [jnp.maximum(
            jnp.dot(soh[i], nrows(hb, i), preferred_element_type=f32)
            + erows(ee, i), 0.0).astype(bf16) for i in R]
        agg = [jnp.dot(doh[i].T, msg[i], preferred_element_type=f32)
               for i in R]
        hl = h + jnp.concatenate(agg, axis=0)
        hl = jnp.dot(
            jnp.maximum(jnp.dot(hl.astype(bf16), Wg1_ref[l],
                                preferred_element_type=f32)
                        + bg1_ref[l][None, :], 0.0).astype(bf16),
            Wg2_ref[l], preferred_element_type=f32) + bg2_ref[l][None, :]
        hl = (hl + h) * _RSBN

        qkv = jnp.dot(hb, Wqkv_ref[l], preferred_element_type=f32) \
            + bqkv_ref[l][None, :]
        heads = [[None] * _HEADS for _ in R]
        for t in range(_HEADS):
            for i in R:
                qi = nrows(qkv, i)
                q = qi[:, t * dh:(t + 1) * dh].astype(bf16)
                k = qi[:, hid + t * dh:hid + (t + 1) * dh].astype(bf16)
                v = qi[:, 2 * hid + t * dh:2 * hid + (t + 1) * dh].astype(bf16)
                sc = jnp.dot(q, k.T, preferred_element_type=f32) + mask
                sc = sc - jnp.max(sc, axis=1, keepdims=True)
                e = jnp.exp(sc)
                rinv = 1.0 / jnp.sum(e, axis=1, keepdims=True)
                heads[i][t] = jnp.dot(e.astype(bf16), v,
                                      preferred_element_type=f32) * rinv
        att = jnp.concatenate(
            [jnp.concatenate(heads[i], axis=1) for i in R], axis=0)
        ha = jnp.dot(att.astype(bf16), Wo_ref[l],
                     preferred_element_type=f32) + bo_ref[l][None, :]
        ha = (ha + h) * _RSBN

        out = hl + ha
        mlp = jnp.dot(
            jnp.maximum(jnp.dot(out.astype(bf16), Wm1_ref[l],
                                preferred_element_type=f32)
                        + bm1_ref[l][None, :], 0.0).astype(bf16),
            Wm2_ref[l], preferred_element_type=f32) + bm2_ref[l][None, :]
        h = (out + mlp) * _RSBN

    pooled = jnp.mean(h.reshape(_GS, _N_PER, hid), axis=1)
    r = jnp.maximum(jnp.dot(pooled, Wh1_ref[...], preferred_element_type=f32)
                    + bh1_ref[...], 0.0)
    y = jnp.sum(r * Wh2t_ref[...], axis=1, keepdims=True)
    out_ref[...] = jnp.broadcast_to(y, out_ref.shape) + bh2_ref[...]


def kernel(x, edge_attr, Wi, bi, We, be, Wg1, bg1, Wg2, bg2, Wqkv, bqkv,
           Wo, bo, Wm1, bm1, Wm2, bm2, Wh1, bh1, Wh2, bh2, edge_index, batch):
    n, d_in = x.shape
    hid = Wi.shape[1]
    n_mol = n // _N_PER
    grid = n_mol // _GS

    src2 = (edge_index[0] % _NPG).astype(jnp.int32).reshape(n_mol, _E_PER)
    dst2 = (edge_index[1] % _NPG).astype(jnp.int32).reshape(n_mol, _E_PER)
    scale = 1.0 / math.sqrt(hid // _HEADS)
    Wqkv = jnp.concatenate([Wqkv[:, :, :hid] * scale, Wqkv[:, :, hid:]],
                           axis=2)
    bqkv = jnp.concatenate([bqkv[:, :hid] * scale, bqkv[:, hid:]], axis=1)

    mol_id = jnp.arange(_NPG, dtype=jnp.int32) // _N_PER
    mask = jnp.where(mol_id[:, None] == mol_id[None, :], 0.0, -1e30)
    mask = mask.astype(jnp.float32)
    eye = jnp.eye(_NPG, dtype=jnp.float32)

    const2 = lambda s: pl.BlockSpec(s, lambda i: (0, 0))
    const3 = lambda s: pl.BlockSpec(s, lambda i: (0, 0, 0))

    out = pl.pallas_call(
        _fused_kernel,
        grid=(grid,),
        in_specs=[
            pl.BlockSpec((_NS * _NPG, d_in), lambda i: (i, 0)),
            pl.BlockSpec((_NS * _EPG, edge_attr.shape[1]), lambda i: (i, 0)),
            pl.BlockSpec((_GS, _E_PER), lambda i: (i, 0)),
            pl.BlockSpec((_GS, _E_PER), lambda i: (i, 0)),
            const2(mask.shape), const2(eye.shape), const2(eye.shape),
            const2(Wi.shape), const2((1, hid)),
            const2(We.shape), const2((1, hid)),
            const3(Wg1.shape), const2(bg1.shape),
            const3(Wg2.shape), const2(bg2.shape),
            const3(Wqkv.shape), const2(bqkv.shape),
            const3(Wo.shape), const2(bo.shape),
            const3(Wm1.shape), const2(bm1.shape),
            const3(Wm2.shape), const2(bm2.shape),
            const2(Wh1.shape), const2((1, Wh1.shape[1])),
            const2((1, Wh2.shape[0])),
            const2((1, hid)),
        ],
        out_specs=pl.BlockSpec((_GS, hid), lambda i: (i, 0)),
        out_shape=jax.ShapeDtypeStruct((n_mol, hid), jnp.float32),
    )(
        x.astype(jnp.bfloat16), edge_attr.astype(jnp.bfloat16),
        src2, dst2, mask, eye, eye.astype(jnp.bfloat16),
        Wi.astype(jnp.bfloat16), bi.reshape(1, -1),
        We.astype(jnp.bfloat16), be.reshape(1, -1),
        Wg1.astype(jnp.bfloat16), bg1, Wg2.astype(jnp.bfloat16), bg2,
        Wqkv.astype(jnp.bfloat16), bqkv, Wo.astype(jnp.bfloat16), bo,
        Wm1.astype(jnp.bfloat16), bm1, Wm2.astype(jnp.bfloat16), bm2,
        Wh1, bh1.reshape(1, -1), Wh2.reshape(1, -1),
        jnp.broadcast_to(bh2.reshape(1, 1), (1, hid)),
    )
    return out[:, 0]
